# Initial kernel scaffold; baseline (speedup 1.0000x reference)
import jax
import jax.numpy as jnp
from jax.experimental import pallas as pl


def kernel(x_num, x_cat, tables, W1, b1, g1, be1, W2, b2, g2, be2, Wpi, bpi, Wmu, bmu):
    raise NotImplementedError("write your pallas kernel here")


# speedup vs baseline: 2.1391x; 2.1391x over previous
import functools

import jax
import jax.numpy as jnp
from jax import lax
from jax.experimental import pallas as pl
from jax.experimental.pallas import tpu as pltpu
from jax.experimental.pallas import tpu_sc as plsc

B = 16384
NUM_DIM = 13
NUM_FIELDS = 26
VOCAB = 100000
EMB_DIM = 16
EPS = 1e-5

R = B * NUM_FIELDS
IDX_COLS = 128
IDX_ROWS = R // IDX_COLS

RB = 1024
NBLK = B // RB
H1 = 256
H2 = 128


def _sc_gather(tables_flat, idx2d):
  info = plsc.get_sparse_core_info()
  nw = info.num_cores * info.num_subcores
  rows_w = R // nw
  irows_w = rows_w // IDX_COLS
  n_chunks = 4
  irows_c = irows_w // n_chunks
  rows_c = rows_w // n_chunks
  mesh = plsc.VectorSubcoreMesh(core_axis_name="c", subcore_axis_name="s")

  @functools.partial(
      pl.kernel,
      mesh=mesh,
      compiler_params=pltpu.CompilerParams(use_tc_tiling_on_sc=False),
      out_type=jax.ShapeDtypeStruct((R, EMB_DIM), jnp.float32),
      scratch_types=[
          pltpu.VMEM((irows_w, IDX_COLS), jnp.int32),
          pltpu.VMEM((rows_c, EMB_DIM), jnp.float32),
          pltpu.SemaphoreType.DMA,
      ],
  )
  def gather_kernel(tab_hbm, idx_hbm, out_hbm, idx_v, data_v, sem):
    wid = lax.axis_index("s") * info.num_cores + lax.axis_index("c")
    pltpu.sync_copy(idx_hbm.at[pl.ds(wid * irows_w, irows_w)], idx_v)
    for c in range(n_chunks):
      handles = []
      for j in range(irows_c):
        handles.append(
            pltpu.async_copy(
                tab_hbm.at[idx_v.at[c * irows_c + j]],
                data_v.at[pl.ds(j * IDX_COLS, IDX_COLS)],
                sem,
            ))
      for h in handles:
        h.wait()
      pltpu.sync_copy(
          data_v, out_hbm.at[pl.ds(wid * rows_w + c * rows_c, rows_c)])

  return gather_kernel(tables_flat, idx2d)


def _mlp_body(xn_ref, e_ref, w1a_ref, w1b_ref, b1_ref, g1_ref, be1_ref,
              w2_ref, b2_ref, g2_ref, be2_ref, wpi_ref, bpi_ref, wmu_ref,
              bmu_ref, pi_ref, mu_ref, h1_s, h2_s, s1, q1, s2, q2):
  p = pl.program_id(0)
  i = pl.program_id(1)
  inv_b = 1.0 / B

  @pl.when(p == 0)
  def _phase0():
    @pl.when(i == 0)
    def _():
      s1[...] = jnp.zeros_like(s1)
      q1[...] = jnp.zeros_like(q1)

    h = (jnp.dot(xn_ref[...], w1a_ref[...], preferred_element_type=jnp.float32)
         + jnp.dot(e_ref[...], w1b_ref[...], preferred_element_type=jnp.float32)
         + b1_ref[...])
    h1_s[pl.ds(i * RB, RB), :] = h
    s1[...] += jnp.sum(h, axis=0, keepdims=True)
    q1[...] += jnp.sum(h * h, axis=0, keepdims=True)

  @pl.when(p == 1)
  def _phase1():
    @pl.when(i == 0)
    def _():
      s2[...] = jnp.zeros_like(s2)
      q2[...] = jnp.zeros_like(q2)

    m = s1[...] * inv_b
    v = q1[...] * inv_b - m * m
    a = g1_ref[...] * lax.rsqrt(v + EPS)
    c = be1_ref[...] - m * a
    h = h1_s[pl.ds(i * RB, RB), :]
    hn = jnp.maximum(h * a + c, 0.0)
    h2 = jnp.dot(hn, w2_ref[...], preferred_element_type=jnp.float32) + b2_ref[...]
    h2_s[pl.ds(i * RB, RB), :] = h2
    s2[...] += jnp.sum(h2, axis=0, keepdims=True)
    q2[...] += jnp.sum(h2 * h2, axis=0, keepdims=True)

  @pl.when(p == 2)
  def _phase2():
    m = s2[...] * inv_b
    v = q2[...] * inv_b - m * m
    a = g2_ref[...] * lax.rsqrt(v + EPS)
    c = be2_ref[...] - m * a
    h = h2_s[pl.ds(i * RB, RB), :]
    hn = jnp.maximum(h * a + c, 0.0)
    logit = jnp.dot(hn, wpi_ref[...], preferred_element_type=jnp.float32) + bpi_ref[...]
    pi_ref[...] = jax.nn.sigmoid(logit)
    mu_ref[...] = jnp.dot(hn, wmu_ref[...], preferred_element_type=jnp.float32) + bmu_ref[...]


def _mlp(x_num, emb, w1a, w1b, b1, g1, be1, w2, b2, g2, be2, wpi, bpi, wmu,
         bmu, interpret=False):
  in_dim_e = NUM_FIELDS * EMB_DIM

  def blk(p, i):
    return (jnp.where(p == 0, i, 0), 0)

  def const(p, i):
    return (0, 0)

  def out_blk(p, i):
    return (i, 0)

  grid = (3, NBLK)
  return pl.pallas_call(
      _mlp_body,
      grid=grid,
      in_specs=[
          pl.BlockSpec((RB, NUM_DIM), blk),
          pl.BlockSpec((RB, in_dim_e), blk),
          pl.BlockSpec((NUM_DIM, H1), const),
          pl.BlockSpec((in_dim_e, H1), const),
          pl.BlockSpec((1, H1), const),
          pl.BlockSpec((1, H1), const),
          pl.BlockSpec((1, H1), const),
          pl.BlockSpec((H1, H2), const),
          pl.BlockSpec((1, H2), const),
          pl.BlockSpec((1, H2), const),
          pl.BlockSpec((1, H2), const),
          pl.BlockSpec((H2, 1), const),
          pl.BlockSpec((1, 1), const),
          pl.BlockSpec((H2, 1), const),
          pl.BlockSpec((1, 1), const),
      ],
      out_specs=[
          pl.BlockSpec((RB, 1), out_blk),
          pl.BlockSpec((RB, 1), out_blk),
      ],
      out_shape=[
          jax.ShapeDtypeStruct((B, 1), jnp.float32),
          jax.ShapeDtypeStruct((B, 1), jnp.float32),
      ],
      scratch_shapes=[
          pltpu.VMEM((B, H1), jnp.float32),
          pltpu.VMEM((B, H2), jnp.float32),
          pltpu.VMEM((1, H1), jnp.float32),
          pltpu.VMEM((1, H1), jnp.float32),
          pltpu.VMEM((1, H2), jnp.float32),
          pltpu.VMEM((1, H2), jnp.float32),
      ],
      compiler_params=pltpu.CompilerParams(
          dimension_semantics=("arbitrary", "arbitrary"),
          vmem_limit_bytes=100 * 1024 * 1024,
      ),
      interpret=interpret,
  )(x_num, emb, w1a, w1b, b1, g1, be1, w2, b2, g2, be2, wpi, bpi, wmu, bmu)


def kernel(x_num, x_cat, tables, W1, b1, g1, be1, W2, b2, g2, be2, Wpi, bpi,
           Wmu, bmu):
  offsets = jnp.arange(NUM_FIELDS, dtype=jnp.int32) * VOCAB
  idx2d = (x_cat + offsets[None, :]).reshape(IDX_ROWS, IDX_COLS)
  tables_flat = tables.reshape(NUM_FIELDS * VOCAB, EMB_DIM)
  emb = _sc_gather(tables_flat, idx2d).reshape(B, NUM_FIELDS * EMB_DIM)

  w1a = W1[:NUM_DIM]
  w1b = W1[NUM_DIM:]
  pi, mu = _mlp(x_num, emb, w1a, w1b, b1.reshape(1, H1), g1.reshape(1, H1),
                be1.reshape(1, H1), W2, b2.reshape(1, H2), g2.reshape(1, H2),
                be2.reshape(1, H2), Wpi, bpi.reshape(1, 1), Wmu,
                bmu.reshape(1, 1))
  return (pi, mu)
